# baseline (device time: 5436 ns/iter reference)
import jax
import jax.numpy as jnp
from jax import lax
from jax.experimental import pallas as pl
from jax.experimental.pallas import tpu as pltpu


def kernel(x):
    m, n = x.shape
    rows, lanes = m // 128, 128

    def body(x_hbm, out_hbm, x_vmem, comm_ref, res_ref,
             in_sem, out_sem, send_sem, recv_sem):
        my_x = lax.axis_index("x")
        my_y = lax.axis_index("y")
        nbr = (my_x, 1 - my_y)

        barrier_sem = pltpu.get_barrier_semaphore()
        pl.semaphore_signal(
            barrier_sem, inc=1, device_id=nbr,
            device_id_type=pl.DeviceIdType.MESH,
        )

        in_copy = pltpu.make_async_copy(x_hbm, x_vmem, in_sem)
        in_copy.start()
        in_copy.wait()
        comm_ref[0, :, :] = jnp.max(x_vmem[:, :], axis=1).reshape(rows, lanes)

        pl.semaphore_wait(barrier_sem, 1)

        rdma = pltpu.make_async_remote_copy(
            src_ref=comm_ref.at[0],
            dst_ref=comm_ref.at[1],
            send_sem=send_sem,
            recv_sem=recv_sem,
            device_id=nbr,
            device_id_type=pl.DeviceIdType.MESH,
        )
        rdma.start()
        rdma.wait_recv()

        res_ref[:] = jnp.maximum(comm_ref[0, :, :], comm_ref[1, :, :]).reshape(m)
        out_copy = pltpu.make_async_copy(res_ref, out_hbm, out_sem)
        out_copy.start()
        out_copy.wait()
        rdma.wait_send()

    dense = pl.pallas_call(
        body,
        out_shape=jax.ShapeDtypeStruct((m,), x.dtype),
        in_specs=[pl.BlockSpec(memory_space=pltpu.MemorySpace.HBM)],
        out_specs=pl.BlockSpec(memory_space=pltpu.MemorySpace.HBM),
        scratch_shapes=[
            pltpu.VMEM((m, n), x.dtype),
            pltpu.VMEM((2, rows, lanes), x.dtype),
            pltpu.VMEM((m,), x.dtype),
            pltpu.SemaphoreType.DMA,
            pltpu.SemaphoreType.DMA,
            pltpu.SemaphoreType.DMA,
            pltpu.SemaphoreType.DMA,
        ],
        compiler_params=pltpu.CompilerParams(collective_id=0),
    )(pltpu.with_memory_space_constraint(x, pltpu.MemorySpace.HBM))
    return dense.reshape(m, 1)


# device time: 5404 ns/iter; 1.0059x vs baseline; 1.0059x over previous
import jax
import jax.numpy as jnp
from jax import lax
from jax.experimental import pallas as pl
from jax.experimental.pallas import tpu as pltpu


def kernel(x):
    m, n = x.shape
    rows, lanes = m // 128, 128

    def body(x_hbm, out_ref, x_vmem, comm_ref,
             in_sem, send_sem, recv_sem):
        my_x = lax.axis_index("x")
        my_y = lax.axis_index("y")
        nbr = (my_x, 1 - my_y)

        barrier_sem = pltpu.get_barrier_semaphore()
        pl.semaphore_signal(
            barrier_sem, inc=1, device_id=nbr,
            device_id_type=pl.DeviceIdType.MESH,
        )

        in_copy = pltpu.make_async_copy(x_hbm, x_vmem, in_sem)
        in_copy.start()
        in_copy.wait()
        comm_ref[0, :, :] = jnp.max(x_vmem[:, :], axis=1).reshape(rows, lanes)

        pl.semaphore_wait(barrier_sem, 1)

        rdma = pltpu.make_async_remote_copy(
            src_ref=comm_ref.at[0],
            dst_ref=comm_ref.at[1],
            send_sem=send_sem,
            recv_sem=recv_sem,
            device_id=nbr,
            device_id_type=pl.DeviceIdType.MESH,
        )
        rdma.start()
        rdma.wait_recv()

        out_ref[:] = jnp.maximum(comm_ref[0, :, :], comm_ref[1, :, :]).reshape(m)
        rdma.wait_send()

    dense = pl.pallas_call(
        body,
        out_shape=jax.ShapeDtypeStruct((m,), x.dtype),
        in_specs=[pl.BlockSpec(memory_space=pltpu.MemorySpace.HBM)],
        out_specs=pl.BlockSpec(memory_space=pltpu.MemorySpace.VMEM),
        scratch_shapes=[
            pltpu.VMEM((m, n), x.dtype),
            pltpu.VMEM((2, rows, lanes), x.dtype),
            pltpu.SemaphoreType.DMA,
            pltpu.SemaphoreType.DMA,
            pltpu.SemaphoreType.DMA,
        ],
        compiler_params=pltpu.CompilerParams(collective_id=0),
    )(pltpu.with_memory_space_constraint(x, pltpu.MemorySpace.HBM))
    return dense.reshape(m, 1)


# device time: 5403 ns/iter; 1.0061x vs baseline; 1.0002x over previous
import jax
import jax.numpy as jnp
from jax import lax
from jax.experimental import pallas as pl
from jax.experimental.pallas import tpu as pltpu

N_CHUNKS = 2


def kernel(x):
    m, n = x.shape
    rows, lanes = m // 128, 128
    mc = m // N_CHUNKS
    rc = rows // N_CHUNKS

    def body(x_hbm, out_ref, x_vmem, comm_ref, in_sems, send_sems, recv_sems):
        my_x = lax.axis_index("x")
        my_y = lax.axis_index("y")
        nbr = (my_x, 1 - my_y)

        barrier_sem = pltpu.get_barrier_semaphore()
        pl.semaphore_signal(
            barrier_sem, inc=1, device_id=nbr,
            device_id_type=pl.DeviceIdType.MESH,
        )

        in_copies = []
        for c in range(N_CHUNKS):
            cp = pltpu.make_async_copy(
                x_hbm.at[pl.ds(c * mc, mc)],
                x_vmem.at[pl.ds(c * mc, mc)],
                in_sems.at[c],
            )
            cp.start()
            in_copies.append(cp)

        rdmas = []
        for c in range(N_CHUNKS):
            in_copies[c].wait()
            comm_ref[0, pl.ds(c * rc, rc), :] = jnp.max(
                x_vmem[pl.ds(c * mc, mc), :], axis=1
            ).reshape(rc, lanes)
            if c == 0:
                pl.semaphore_wait(barrier_sem, 1)
            rdma = pltpu.make_async_remote_copy(
                src_ref=comm_ref.at[0, pl.ds(c * rc, rc)],
                dst_ref=comm_ref.at[1, pl.ds(c * rc, rc)],
                send_sem=send_sems.at[c],
                recv_sem=recv_sems.at[c],
                device_id=nbr,
                device_id_type=pl.DeviceIdType.MESH,
            )
            rdma.start()
            rdmas.append(rdma)

        for c in range(N_CHUNKS):
            rdmas[c].wait_recv()
            out_ref[pl.ds(c * rc * lanes, rc * lanes)] = jnp.maximum(
                comm_ref[0, pl.ds(c * rc, rc), :],
                comm_ref[1, pl.ds(c * rc, rc), :],
            ).reshape(rc * lanes)

        for c in range(N_CHUNKS):
            rdmas[c].wait_send()

    dense = pl.pallas_call(
        body,
        out_shape=jax.ShapeDtypeStruct((m,), x.dtype),
        in_specs=[pl.BlockSpec(memory_space=pltpu.MemorySpace.HBM)],
        out_specs=pl.BlockSpec(memory_space=pltpu.MemorySpace.VMEM),
        scratch_shapes=[
            pltpu.VMEM((m, n), x.dtype),
            pltpu.VMEM((2, rows, lanes), x.dtype),
            pltpu.SemaphoreType.DMA((N_CHUNKS,)),
            pltpu.SemaphoreType.DMA((N_CHUNKS,)),
            pltpu.SemaphoreType.DMA((N_CHUNKS,)),
        ],
        compiler_params=pltpu.CompilerParams(collective_id=0),
    )(pltpu.with_memory_space_constraint(x, pltpu.MemorySpace.HBM))
    return dense.reshape(m, 1)
